# baseline (speedup 1.0000x reference)
import functools

import jax
import jax.numpy as jnp
from jax import lax
from jax.experimental import pallas as pl
from jax.experimental.pallas import tpu as pltpu
from jax.experimental.pallas import tpu_sc as plsc

_B, _L, _D = 4096, 200, 64
_NC, _NS = 2, 16
_NW = _NC * _NS
_ROWS_PER_W = _B // _NW
_C0, _C1 = 128, 72
_LPAD = 208


def _sc_pool(pair_idx, base_off, table2):
  mesh = plsc.VectorSubcoreMesh(core_axis_name="c", subcore_axis_name="s")

  @functools.partial(
      pl.kernel,
      out_type=jax.ShapeDtypeStruct((_B * _D,), jnp.float32),
      mesh=mesh,
      scratch_types=[
          pltpu.VMEM((_ROWS_PER_W * _L,), jnp.int32),
          pltpu.VMEM((_ROWS_PER_W * _LPAD,), jnp.int32),
          pltpu.VMEM((_L, 2 * _D), jnp.float32),
          pltpu.VMEM((_L, 2 * _D), jnp.float32),
          pltpu.VMEM((_ROWS_PER_W * _D,), jnp.float32),
          pltpu.SemaphoreType.DMA,
          pltpu.SemaphoreType.DMA,
      ],
  )
  def pool(idx_hbm, lsb_hbm, table_hbm, out_hbm, idx_v, lsb_v, buf_a, buf_b,
           out_v, sem_a, sem_b):
    c = lax.axis_index("c")
    s = lax.axis_index("s")
    w = c * _NS + s

    pltpu.sync_copy(idx_hbm.at[pl.ds(w * (_ROWS_PER_W * _L), _ROWS_PER_W * _L)],
                    idx_v)
    pltpu.sync_copy(
        lsb_hbm.at[pl.ds(w * (_ROWS_PER_W * _LPAD), _ROWS_PER_W * _LPAD)],
        lsb_v)

    def gather_cps(r, buf, sem):
      cp0 = pltpu.make_async_copy(table_hbm.at[idx_v.at[pl.ds(r * _L, _C0)]],
                                  buf.at[pl.ds(0, _C0)], sem)
      cp1 = pltpu.make_async_copy(
          table_hbm.at[idx_v.at[pl.ds(r * _L + _C0, _C1)]],
          buf.at[pl.ds(_C0, _C1)], sem)
      return cp0, cp1

    def start(r, buf, sem):
      cp0, cp1 = gather_cps(r, buf, sem)
      cp0.start()
      cp1.start()

    def wait(r, buf, sem):
      cp0, cp1 = gather_cps(r, buf, sem)
      cp0.wait()
      cp1.wait()

    def reduce_row(buf, r):
      def step(j, base, acc):
        a0, a1, a2, a3 = acc
        a0 = a0 + buf[j, pl.ds(base, 16)]
        a1 = a1 + buf[j, pl.ds(base + 16, 16)]
        a2 = a2 + buf[j, pl.ds(base + 32, 16)]
        a3 = a3 + buf[j, pl.ds(base + 48, 16)]
        return (a0, a1, a2, a3)

      def blk(jc, acc):
        j0 = jc * 16
        bases = lsb_v[pl.ds(r * _LPAD + j0, 16)]
        for t in range(16):
          acc = step(j0 + t, bases[t], acc)
        return acc

      z = jnp.zeros((16,), jnp.float32)
      acc = lax.fori_loop(0, _L // 16, blk, (z, z, z, z))
      j0 = (_L // 16) * 16
      bases = lsb_v[pl.ds(r * _LPAD + j0, 16)]
      for t in range(_L - j0):
        acc = step(j0 + t, bases[t], acc)
      a0, a1, a2, a3 = acc
      out_v[pl.ds(r * _D, 16)] = a0
      out_v[pl.ds(r * _D + 16, 16)] = a1
      out_v[pl.ds(r * _D + 32, 16)] = a2
      out_v[pl.ds(r * _D + 48, 16)] = a3

    start(0, buf_a, sem_a)

    @pl.loop(0, _ROWS_PER_W - 2, step=2)
    def _(r):
      start(r + 1, buf_b, sem_b)
      wait(r, buf_a, sem_a)
      reduce_row(buf_a, r)
      start(r + 2, buf_a, sem_a)
      wait(r + 1, buf_b, sem_b)
      reduce_row(buf_b, r + 1)

    r_last = _ROWS_PER_W - 2
    start(r_last + 1, buf_b, sem_b)
    wait(r_last, buf_a, sem_a)
    reduce_row(buf_a, r_last)
    wait(r_last + 1, buf_b, sem_b)
    reduce_row(buf_b, r_last + 1)

    pltpu.sync_copy(
        out_v, out_hbm.at[pl.ds(w * (_ROWS_PER_W * _D), _ROWS_PER_W * _D)])

  return pool(pair_idx, base_off, table2)


def _mlp_body(enc_ref, tl_ref, w1_ref, b1_ref, w2_ref, b2_ref, out_ref):
  enc = enc_ref[...] * (1.0 / tl_ref[...])
  h = jnp.dot(enc, w1_ref[...], preferred_element_type=jnp.float32)
  h = jnp.maximum(h + b1_ref[...], 0.0)
  out = jnp.dot(h, w2_ref[...], preferred_element_type=jnp.float32)
  out_ref[...] = out + b2_ref[...]


def _mlp(encoded, text_len, w1t, b1, w2t, b2):
  bb = 512
  h = w1t.shape[1]
  cc = w2t.shape[1]
  return pl.pallas_call(
      _mlp_body,
      grid=(_B // bb,),
      in_specs=[
          pl.BlockSpec((bb, _D), lambda i: (i, 0)),
          pl.BlockSpec((bb, 1), lambda i: (i, 0)),
          pl.BlockSpec((_D, h), lambda i: (0, 0)),
          pl.BlockSpec((1, h), lambda i: (0, 0)),
          pl.BlockSpec((h, cc), lambda i: (0, 0)),
          pl.BlockSpec((1, cc), lambda i: (0, 0)),
      ],
      out_specs=pl.BlockSpec((bb, cc), lambda i: (i, 0)),
      out_shape=jax.ShapeDtypeStruct((_B, cc), jnp.float32),
  )(encoded, text_len.reshape(_B, 1), w1t, b1.reshape(1, h), w2t,
    b2.reshape(1, cc))


def kernel(input_text, text_len, table, W1, b1, W2, b2):
  v = table.shape[0]
  table2 = table.reshape(v // 2, 2 * _D)
  pair_idx = (input_text >> 1).reshape(_B * _L)
  base_off = jnp.pad((input_text & 1) << 6,
                     ((0, 0), (0, _LPAD - _L))).reshape(_B * _LPAD)
  encoded = _sc_pool(pair_idx, base_off, table2).reshape(_B, _D)
  return _mlp(encoded, text_len, W1.T, b1, W2.T, b2)


# speedup vs baseline: 1.4838x; 1.4838x over previous
import functools

import jax
import jax.numpy as jnp
from jax import lax
from jax.experimental import pallas as pl
from jax.experimental.pallas import tpu as pltpu
from jax.experimental.pallas import tpu_sc as plsc

_B, _L, _D = 4096, 200, 64
_NC, _NS = 2, 16
_NW = _NC * _NS
_ROWS_PER_W = _B // _NW
_HALF = _L // 2


def _sc_pool(idx_flat, table):
  mesh = plsc.VectorSubcoreMesh(core_axis_name="c", subcore_axis_name="s")

  @functools.partial(
      pl.kernel,
      out_type=jax.ShapeDtypeStruct((_B, _D), jnp.float32),
      mesh=mesh,
      compiler_params=pltpu.CompilerParams(use_tc_tiling_on_sc=False),
      scratch_types=[
          pltpu.VMEM((2 * _ROWS_PER_W, _HALF), jnp.int32),
          pltpu.VMEM((_L, _D), jnp.float32),
          pltpu.VMEM((_L, _D), jnp.float32),
          pltpu.VMEM((_ROWS_PER_W, _D), jnp.float32),
          pltpu.SemaphoreType.DMA,
          pltpu.SemaphoreType.DMA,
      ],
  )
  def pool(idx_hbm, table_hbm, out_hbm, idx_v, buf_a, buf_b, out_v, sem_a,
           sem_b):
    c = lax.axis_index("c")
    s = lax.axis_index("s")
    w = c * _NS + s

    pltpu.sync_copy(idx_hbm.at[pl.ds(w * (2 * _ROWS_PER_W), 2 * _ROWS_PER_W)],
                    idx_v)

    def gather_cps(r, buf, sem):
      cp0 = pltpu.make_async_copy(table_hbm.at[idx_v.at[2 * r]],
                                  buf.at[pl.ds(0, _HALF)], sem)
      cp1 = pltpu.make_async_copy(table_hbm.at[idx_v.at[2 * r + 1]],
                                  buf.at[pl.ds(_HALF, _HALF)], sem)
      return cp0, cp1

    def start(r, buf, sem):
      cp0, cp1 = gather_cps(r, buf, sem)
      cp0.start()
      cp1.start()

    def wait(r, buf, sem):
      cp0, cp1 = gather_cps(r, buf, sem)
      cp0.wait()
      cp1.wait()

    def reduce_row(buf, r):
      def body(j, acc):
        a0, a1, a2, a3 = acc
        a0 = a0 + buf[j, pl.ds(0, 16)]
        a1 = a1 + buf[j, pl.ds(16, 16)]
        a2 = a2 + buf[j, pl.ds(32, 16)]
        a3 = a3 + buf[j, pl.ds(48, 16)]
        return (a0, a1, a2, a3)

      z = jnp.zeros((16,), jnp.float32)
      a0, a1, a2, a3 = lax.fori_loop(0, _L, body, (z, z, z, z))
      out_v[r, pl.ds(0, 16)] = a0
      out_v[r, pl.ds(16, 16)] = a1
      out_v[r, pl.ds(32, 16)] = a2
      out_v[r, pl.ds(48, 16)] = a3

    start(0, buf_a, sem_a)

    @pl.loop(0, _ROWS_PER_W - 2, step=2)
    def _(r):
      start(r + 1, buf_b, sem_b)
      wait(r, buf_a, sem_a)
      reduce_row(buf_a, r)
      start(r + 2, buf_a, sem_a)
      wait(r + 1, buf_b, sem_b)
      reduce_row(buf_b, r + 1)

    r_last = _ROWS_PER_W - 2
    start(r_last + 1, buf_b, sem_b)
    wait(r_last, buf_a, sem_a)
    reduce_row(buf_a, r_last)
    wait(r_last + 1, buf_b, sem_b)
    reduce_row(buf_b, r_last + 1)

    pltpu.sync_copy(out_v, out_hbm.at[pl.ds(w * _ROWS_PER_W, _ROWS_PER_W)])

  return pool(idx_flat, table)


def _mlp_body(enc_ref, tl_ref, w1_ref, b1_ref, w2_ref, b2_ref, out_ref):
  enc = enc_ref[...] * (1.0 / tl_ref[...])
  h = jnp.dot(enc, w1_ref[...], preferred_element_type=jnp.float32)
  h = jnp.maximum(h + b1_ref[...], 0.0)
  out = jnp.dot(h, w2_ref[...], preferred_element_type=jnp.float32)
  out_ref[...] = out + b2_ref[...]


def _mlp(encoded, text_len, w1t, b1, w2t, b2):
  bb = 512
  h = w1t.shape[1]
  cc = w2t.shape[1]
  return pl.pallas_call(
      _mlp_body,
      grid=(_B // bb,),
      in_specs=[
          pl.BlockSpec((bb, _D), lambda i: (i, 0)),
          pl.BlockSpec((bb, 1), lambda i: (i, 0)),
          pl.BlockSpec((_D, h), lambda i: (0, 0)),
          pl.BlockSpec((1, h), lambda i: (0, 0)),
          pl.BlockSpec((h, cc), lambda i: (0, 0)),
          pl.BlockSpec((1, cc), lambda i: (0, 0)),
      ],
      out_specs=pl.BlockSpec((bb, cc), lambda i: (i, 0)),
      out_shape=jax.ShapeDtypeStruct((_B, cc), jnp.float32),
  )(encoded, text_len.reshape(_B, 1), w1t, b1.reshape(1, h), w2t,
    b2.reshape(1, cc))


def kernel(input_text, text_len, table, W1, b1, W2, b2):
  idx_flat = input_text.reshape(_B * _L // _HALF, _HALF)
  encoded = _sc_pool(idx_flat, table)
  return _mlp(encoded, text_len, W1.T, b1, W2.T, b2)


# speedup vs baseline: 2.2567x; 1.5209x over previous
import functools

import jax
import jax.numpy as jnp
from jax import lax
from jax.experimental import pallas as pl
from jax.experimental.pallas import tpu as pltpu
from jax.experimental.pallas import tpu_sc as plsc

_B, _L, _D, _H = 4096, 200, 64, 128
_NC, _NS = 2, 16
_NW = _NC * _NS
_ROWS_PER_W = _B // _NW
_C0, _C1 = 128, 72
_GM = 8192


def _g_matmul(table_t, w1t):
  v = table_t.shape[1]

  def body(t_ref, w_ref, g_ref):
    g_ref[...] = lax.dot_general(t_ref[...], w_ref[...],
                                 (((0,), (0,)), ((), ())),
                                 preferred_element_type=jnp.float32)

  return pl.pallas_call(
      body,
      grid=((v + _GM - 1) // _GM,),
      in_specs=[
          pl.BlockSpec((_D, _GM), lambda i: (0, i)),
          pl.BlockSpec((_D, _H), lambda i: (0, 0)),
      ],
      out_specs=pl.BlockSpec((_GM, _H), lambda i: (i, 0)),
      out_shape=jax.ShapeDtypeStruct((v, _H), jnp.float32),
  )(table_t, w1t)


def _sc_pool(idx_flat, g):
  mesh = plsc.VectorSubcoreMesh(core_axis_name="c", subcore_axis_name="s")

  @functools.partial(
      pl.kernel,
      out_type=jax.ShapeDtypeStruct((_B * _H,), jnp.float32),
      mesh=mesh,
      scratch_types=[
          pltpu.VMEM((_ROWS_PER_W * _L,), jnp.int32),
          pltpu.VMEM((_L, _H), jnp.float32),
          pltpu.VMEM((_L, _H), jnp.float32),
          pltpu.VMEM((_ROWS_PER_W * _H,), jnp.float32),
          pltpu.SemaphoreType.DMA,
          pltpu.SemaphoreType.DMA,
      ],
  )
  def pool(idx_hbm, g_hbm, out_hbm, idx_v, buf_a, buf_b, out_v, sem_a, sem_b):
    c = lax.axis_index("c")
    s = lax.axis_index("s")
    w = c * _NS + s

    pltpu.sync_copy(idx_hbm.at[pl.ds(w * (_ROWS_PER_W * _L), _ROWS_PER_W * _L)],
                    idx_v)

    def gather_cps(r, buf, sem):
      cp0 = pltpu.make_async_copy(g_hbm.at[idx_v.at[pl.ds(r * _L, _C0)]],
                                  buf.at[pl.ds(0, _C0)], sem)
      cp1 = pltpu.make_async_copy(g_hbm.at[idx_v.at[pl.ds(r * _L + _C0, _C1)]],
                                  buf.at[pl.ds(_C0, _C1)], sem)
      return cp0, cp1

    def start(r, buf, sem):
      cp0, cp1 = gather_cps(r, buf, sem)
      cp0.start()
      cp1.start()

    def wait(r, buf, sem):
      cp0, cp1 = gather_cps(r, buf, sem)
      cp0.wait()
      cp1.wait()

    def reduce_row(buf, r):
      def body(j, acc):
        return tuple(acc[k] + buf[j, pl.ds(16 * k, 16)] for k in range(8))

      z = jnp.zeros((16,), jnp.float32)
      acc = lax.fori_loop(0, _L, body, (z,) * 8)
      for k in range(8):
        out_v[pl.ds(r * _H + 16 * k, 16)] = acc[k]

    start(0, buf_a, sem_a)

    @pl.loop(0, _ROWS_PER_W - 2, step=2)
    def _(r):
      start(r + 1, buf_b, sem_b)
      wait(r, buf_a, sem_a)
      reduce_row(buf_a, r)
      start(r + 2, buf_a, sem_a)
      wait(r + 1, buf_b, sem_b)
      reduce_row(buf_b, r + 1)

    r_last = _ROWS_PER_W - 2
    start(r_last + 1, buf_b, sem_b)
    wait(r_last, buf_a, sem_a)
    reduce_row(buf_a, r_last)
    wait(r_last + 1, buf_b, sem_b)
    reduce_row(buf_b, r_last + 1)

    pltpu.sync_copy(
        out_v, out_hbm.at[pl.ds(w * (_ROWS_PER_W * _H), _ROWS_PER_W * _H)])

  return pool(idx_flat, g)


def _mlp_body(enc_ref, tl_ref, b1_ref, w2_ref, b2_ref, out_ref):
  h = enc_ref[...] * (1.0 / tl_ref[...]) + b1_ref[...]
  h = jnp.maximum(h, 0.0)
  out = jnp.dot(h, w2_ref[...], preferred_element_type=jnp.float32)
  out_ref[...] = out + b2_ref[...]


def _mlp(pooled, text_len, b1, w2t, b2):
  bb = 512
  cc = w2t.shape[1]
  return pl.pallas_call(
      _mlp_body,
      grid=(_B // bb,),
      in_specs=[
          pl.BlockSpec((bb, _H), lambda i: (i, 0)),
          pl.BlockSpec((bb, 1), lambda i: (i, 0)),
          pl.BlockSpec((1, _H), lambda i: (0, 0)),
          pl.BlockSpec((_H, cc), lambda i: (0, 0)),
          pl.BlockSpec((1, cc), lambda i: (0, 0)),
      ],
      out_specs=pl.BlockSpec((bb, cc), lambda i: (i, 0)),
      out_shape=jax.ShapeDtypeStruct((_B, cc), jnp.float32),
  )(pooled, text_len.reshape(_B, 1), b1.reshape(1, _H), w2t,
    b2.reshape(1, cc))


def kernel(input_text, text_len, table, W1, b1, W2, b2):
  g = _g_matmul(table.T, W1.T)
  idx_flat = input_text.reshape(_B * _L)
  pooled = _sc_pool(idx_flat, g).reshape(_B, _H)
  return _mlp(pooled, text_len, b1, W2.T, b2)


# speedup vs baseline: 2.3272x; 1.0312x over previous
import functools

import jax
import jax.numpy as jnp
from jax import lax
from jax.experimental import pallas as pl
from jax.experimental.pallas import tpu as pltpu
from jax.experimental.pallas import tpu_sc as plsc

_B, _L, _D, _H = 4096, 200, 64, 128
_NC, _NS = 2, 16
_NW = _NC * _NS
_ROWS_PER_W = _B // _NW
_C0, _C1 = 128, 72
_GM = 8192


def _g_matmul(table_t, w1t):
  v = table_t.shape[1]

  def body(t_ref, w_ref, g_ref):
    g_ref[...] = lax.dot_general(t_ref[...], w_ref[...],
                                 (((0,), (0,)), ((), ())),
                                 preferred_element_type=jnp.float32)

  return pl.pallas_call(
      body,
      grid=((v + _GM - 1) // _GM,),
      in_specs=[
          pl.BlockSpec((_D, _GM), lambda i: (0, i)),
          pl.BlockSpec((_D, _H), lambda i: (0, 0)),
      ],
      out_specs=pl.BlockSpec((_GM, _H), lambda i: (i, 0)),
      out_shape=jax.ShapeDtypeStruct((v, _H), jnp.float32),
  )(table_t, w1t)


def _sc_pool(idx_flat, g):
  mesh = plsc.VectorSubcoreMesh(core_axis_name="c", subcore_axis_name="s")

  @functools.partial(
      pl.kernel,
      out_type=jax.ShapeDtypeStruct((_B * _H,), jnp.float32),
      mesh=mesh,
      scratch_types=[
          pltpu.VMEM((_ROWS_PER_W * _L,), jnp.int32),
          pltpu.VMEM((_L, _H), jnp.float32),
          pltpu.VMEM((_L, _H), jnp.float32),
          pltpu.VMEM((_ROWS_PER_W * _H,), jnp.float32),
          pltpu.SemaphoreType.DMA,
          pltpu.SemaphoreType.DMA,
      ],
  )
  def pool(idx_hbm, g_hbm, out_hbm, idx_v, buf_a, buf_b, out_v, sem_a, sem_b):
    c = lax.axis_index("c")
    s = lax.axis_index("s")
    w = c * _NS + s

    pltpu.sync_copy(idx_hbm.at[pl.ds(w * (_ROWS_PER_W * _L), _ROWS_PER_W * _L)],
                    idx_v)

    def gather_cps(r, buf, sem):
      cp0 = pltpu.make_async_copy(g_hbm.at[idx_v.at[pl.ds(r * _L, _C0)]],
                                  buf.at[pl.ds(0, _C0)], sem)
      cp1 = pltpu.make_async_copy(g_hbm.at[idx_v.at[pl.ds(r * _L + _C0, _C1)]],
                                  buf.at[pl.ds(_C0, _C1)], sem)
      return cp0, cp1

    def start(r, buf, sem):
      cp0, cp1 = gather_cps(r, buf, sem)
      cp0.start()
      cp1.start()

    def wait(r, buf, sem):
      cp0, cp1 = gather_cps(r, buf, sem)
      cp0.wait()
      cp1.wait()

    def reduce_row(buf, r):
      def body(jj, acc):
        j = jj * 4
        for t in range(4):
          acc = tuple(acc[k] + buf[j + t, pl.ds(16 * k, 16)] for k in range(8))
        return acc

      z = jnp.zeros((16,), jnp.float32)
      acc = lax.fori_loop(0, _L // 4, body, (z,) * 8)
      for k in range(8):
        out_v[pl.ds(r * _H + 16 * k, 16)] = acc[k]

    start(0, buf_a, sem_a)

    @pl.loop(0, _ROWS_PER_W - 2, step=2)
    def _(r):
      start(r + 1, buf_b, sem_b)
      wait(r, buf_a, sem_a)
      reduce_row(buf_a, r)
      start(r + 2, buf_a, sem_a)
      wait(r + 1, buf_b, sem_b)
      reduce_row(buf_b, r + 1)

    r_last = _ROWS_PER_W - 2
    start(r_last + 1, buf_b, sem_b)
    wait(r_last, buf_a, sem_a)
    reduce_row(buf_a, r_last)
    wait(r_last + 1, buf_b, sem_b)
    reduce_row(buf_b, r_last + 1)

    pltpu.sync_copy(
        out_v, out_hbm.at[pl.ds(w * (_ROWS_PER_W * _H), _ROWS_PER_W * _H)])

  return pool(idx_flat, g)


def _mlp_body(enc_ref, tl_ref, b1_ref, w2_ref, b2_ref, out_ref):
  h = enc_ref[...] * (1.0 / tl_ref[...]) + b1_ref[...]
  h = jnp.maximum(h, 0.0)
  out = lax.dot_general(w2_ref[...], h, (((1,), (1,)), ((), ())),
                        preferred_element_type=jnp.float32)
  out_ref[...] = out + b2_ref[...]


def _mlp(pooled, text_len, b1, w2, b2):
  bb = 512
  cc = w2.shape[0]
  return pl.pallas_call(
      _mlp_body,
      grid=(_B // bb,),
      in_specs=[
          pl.BlockSpec((bb, _H), lambda i: (i, 0)),
          pl.BlockSpec((bb, 1), lambda i: (i, 0)),
          pl.BlockSpec((1, _H), lambda i: (0, 0)),
          pl.BlockSpec((cc, _H), lambda i: (0, 0)),
          pl.BlockSpec((cc, 1), lambda i: (0, 0)),
      ],
      out_specs=pl.BlockSpec((cc, bb), lambda i: (0, i)),
      out_shape=jax.ShapeDtypeStruct((cc, _B), jnp.float32),
  )(pooled, text_len.reshape(_B, 1), b1.reshape(1, _H), w2,
    b2.reshape(cc, 1))


def kernel(input_text, text_len, table, W1, b1, W2, b2):
  g = _g_matmul(table.T, W1.T)
  idx_flat = input_text.reshape(_B * _L)
  pooled = _sc_pool(idx_flat, g).reshape(_B, _H)
  return _mlp(pooled, text_len, b1, W2, b2).T


# speedup vs baseline: 2.4573x; 1.0559x over previous
import functools

import jax
import jax.numpy as jnp
from jax import lax
from jax.experimental import pallas as pl
from jax.experimental.pallas import tpu as pltpu
from jax.experimental.pallas import tpu_sc as plsc

_B, _L, _D, _H = 4096, 200, 64, 128
_NC, _NS = 2, 16
_NW = _NC * _NS
_ROWS_PER_W = _B // _NW
_C0, _C1 = 128, 72
_GM = 16384


def _g_matmul(table_t, w1t):
  v = table_t.shape[1]

  def body(t_ref, w_ref, g_ref):
    g_ref[...] = lax.dot_general(t_ref[...], w_ref[...],
                                 (((0,), (0,)), ((), ())),
                                 preferred_element_type=jnp.float32)

  return pl.pallas_call(
      body,
      grid=((v + _GM - 1) // _GM,),
      in_specs=[
          pl.BlockSpec((_D, _GM), lambda i: (0, i)),
          pl.BlockSpec((_D, _H), lambda i: (0, 0)),
      ],
      out_specs=pl.BlockSpec((_GM, _H), lambda i: (i, 0)),
      out_shape=jax.ShapeDtypeStruct((v, _H), jnp.float32),
  )(table_t, w1t)


def _sc_pool(idx_flat, g):
  mesh = plsc.VectorSubcoreMesh(core_axis_name="c", subcore_axis_name="s")

  @functools.partial(
      pl.kernel,
      out_type=jax.ShapeDtypeStruct((_B * _H,), jnp.float32),
      mesh=mesh,
      scratch_types=[
          pltpu.VMEM((_ROWS_PER_W * _L,), jnp.int32),
          pltpu.VMEM((_L, _H), jnp.float32),
          pltpu.VMEM((_L, _H), jnp.float32),
          pltpu.VMEM((_ROWS_PER_W * _H,), jnp.float32),
          pltpu.SemaphoreType.DMA,
          pltpu.SemaphoreType.DMA,
      ],
  )
  def pool(idx_hbm, g_hbm, out_hbm, idx_v, buf_a, buf_b, out_v, sem_a, sem_b):
    c = lax.axis_index("c")
    s = lax.axis_index("s")
    w = c * _NS + s

    pltpu.sync_copy(idx_hbm.at[pl.ds(w * (_ROWS_PER_W * _L), _ROWS_PER_W * _L)],
                    idx_v)

    def gather_cps(r, buf, sem):
      cp0 = pltpu.make_async_copy(g_hbm.at[idx_v.at[pl.ds(r * _L, _C0)]],
                                  buf.at[pl.ds(0, _C0)], sem)
      cp1 = pltpu.make_async_copy(g_hbm.at[idx_v.at[pl.ds(r * _L + _C0, _C1)]],
                                  buf.at[pl.ds(_C0, _C1)], sem)
      return cp0, cp1

    def start(r, buf, sem):
      cp0, cp1 = gather_cps(r, buf, sem)
      cp0.start()
      cp1.start()

    def wait(r, buf, sem):
      cp0, cp1 = gather_cps(r, buf, sem)
      cp0.wait()
      cp1.wait()

    def reduce_row(buf, r):
      def body(jj, acc):
        j = jj * 4
        for t in range(4):
          acc = tuple(acc[k] + buf[j + t, pl.ds(16 * k, 16)] for k in range(8))
        return acc

      z = jnp.zeros((16,), jnp.float32)
      acc = lax.fori_loop(0, _L // 4, body, (z,) * 8)
      for k in range(8):
        out_v[pl.ds(r * _H + 16 * k, 16)] = acc[k]

    start(0, buf_a, sem_a)

    @pl.loop(0, _ROWS_PER_W - 2, step=2)
    def _(r):
      start(r + 1, buf_b, sem_b)
      wait(r, buf_a, sem_a)
      reduce_row(buf_a, r)
      start(r + 2, buf_a, sem_a)
      wait(r + 1, buf_b, sem_b)
      reduce_row(buf_b, r + 1)

    r_last = _ROWS_PER_W - 2
    start(r_last + 1, buf_b, sem_b)
    wait(r_last, buf_a, sem_a)
    reduce_row(buf_a, r_last)
    wait(r_last + 1, buf_b, sem_b)
    reduce_row(buf_b, r_last + 1)

    pltpu.sync_copy(
        out_v, out_hbm.at[pl.ds(w * (_ROWS_PER_W * _H), _ROWS_PER_W * _H)])

  return pool(idx_flat, g)


def _mlp_body(enc_ref, tl_ref, b1_ref, w2_ref, b2_ref, out_ref):
  h = enc_ref[...] * (1.0 / tl_ref[...]) + b1_ref[...]
  h = jnp.maximum(h, 0.0)
  out = lax.dot_general(w2_ref[...], h, (((1,), (1,)), ((), ())),
                        preferred_element_type=jnp.float32)
  out_ref[...] = out + b2_ref[...]


def _mlp(pooled, text_len, b1, w2, b2):
  bb = 512
  cc = w2.shape[0]
  return pl.pallas_call(
      _mlp_body,
      grid=(_B // bb,),
      in_specs=[
          pl.BlockSpec((bb, _H), lambda i: (i, 0)),
          pl.BlockSpec((bb, 1), lambda i: (i, 0)),
          pl.BlockSpec((1, _H), lambda i: (0, 0)),
          pl.BlockSpec((cc, _H), lambda i: (0, 0)),
          pl.BlockSpec((cc, 1), lambda i: (0, 0)),
      ],
      out_specs=pl.BlockSpec((cc, bb), lambda i: (0, i)),
      out_shape=jax.ShapeDtypeStruct((cc, _B), jnp.float32),
  )(pooled, text_len.reshape(_B, 1), b1.reshape(1, _H), w2,
    b2.reshape(cc, 1))


def kernel(input_text, text_len, table, W1, b1, W2, b2):
  g = _g_matmul(table.T, W1.T)
  idx_flat = input_text.reshape(_B * _L)
  pooled = _sc_pool(idx_flat, g).reshape(_B, _H)
  return _mlp(pooled, text_len, b1, W2, b2).T


# speedup vs baseline: 2.4839x; 1.0108x over previous
import functools

import jax
import jax.numpy as jnp
from jax import lax
from jax.experimental import pallas as pl
from jax.experimental.pallas import tpu as pltpu
from jax.experimental.pallas import tpu_sc as plsc

_B, _L, _D, _H = 4096, 200, 64, 128
_NC, _NS = 2, 16
_NW = _NC * _NS
_ROWS_PER_W = _B // _NW
_C0, _C1 = 128, 72
_GM = 32768


def _g_matmul(table_t, w1t):
  v = table_t.shape[1]

  def body(t_ref, w_ref, g_ref):
    g_ref[...] = lax.dot_general(t_ref[...], w_ref[...],
                                 (((0,), (0,)), ((), ())),
                                 preferred_element_type=jnp.float32)

  return pl.pallas_call(
      body,
      grid=((v + _GM - 1) // _GM,),
      in_specs=[
          pl.BlockSpec((_D, _GM), lambda i: (0, i)),
          pl.BlockSpec((_D, _H), lambda i: (0, 0)),
      ],
      out_specs=pl.BlockSpec((_GM, _H), lambda i: (i, 0)),
      out_shape=jax.ShapeDtypeStruct((v, _H), jnp.float32),
  )(table_t, w1t)


def _sc_pool(idx_flat, g):
  mesh = plsc.VectorSubcoreMesh(core_axis_name="c", subcore_axis_name="s")

  @functools.partial(
      pl.kernel,
      out_type=jax.ShapeDtypeStruct((_B * _H,), jnp.float32),
      mesh=mesh,
      scratch_types=[
          pltpu.VMEM((_ROWS_PER_W * _L,), jnp.int32),
          pltpu.VMEM((_L, _H), jnp.float32),
          pltpu.VMEM((_L, _H), jnp.float32),
          pltpu.VMEM((_ROWS_PER_W * _H,), jnp.float32),
          pltpu.SemaphoreType.DMA,
          pltpu.SemaphoreType.DMA,
      ],
  )
  def pool(idx_hbm, g_hbm, out_hbm, idx_v, buf_a, buf_b, out_v, sem_a, sem_b):
    c = lax.axis_index("c")
    s = lax.axis_index("s")
    w = c * _NS + s

    pltpu.sync_copy(idx_hbm.at[pl.ds(w * (_ROWS_PER_W * _L), _ROWS_PER_W * _L)],
                    idx_v)

    def gather_cps(r, buf, sem):
      cp0 = pltpu.make_async_copy(g_hbm.at[idx_v.at[pl.ds(r * _L, _C0)]],
                                  buf.at[pl.ds(0, _C0)], sem)
      cp1 = pltpu.make_async_copy(g_hbm.at[idx_v.at[pl.ds(r * _L + _C0, _C1)]],
                                  buf.at[pl.ds(_C0, _C1)], sem)
      return cp0, cp1

    def start(r, buf, sem):
      cp0, cp1 = gather_cps(r, buf, sem)
      cp0.start()
      cp1.start()

    def wait(r, buf, sem):
      cp0, cp1 = gather_cps(r, buf, sem)
      cp0.wait()
      cp1.wait()

    def reduce_row(buf, r):
      def body(jj, acc):
        j = jj * 4
        for t in range(4):
          acc = tuple(acc[k] + buf[j + t, pl.ds(16 * k, 16)] for k in range(8))
        return acc

      z = jnp.zeros((16,), jnp.float32)
      acc = lax.fori_loop(0, _L // 4, body, (z,) * 8)
      for k in range(8):
        out_v[pl.ds(r * _H + 16 * k, 16)] = acc[k]

    start(0, buf_a, sem_a)

    @pl.loop(0, _ROWS_PER_W - 2, step=2)
    def _(r):
      start(r + 1, buf_b, sem_b)
      wait(r, buf_a, sem_a)
      reduce_row(buf_a, r)
      start(r + 2, buf_a, sem_a)
      wait(r + 1, buf_b, sem_b)
      reduce_row(buf_b, r + 1)

    r_last = _ROWS_PER_W - 2
    start(r_last + 1, buf_b, sem_b)
    wait(r_last, buf_a, sem_a)
    reduce_row(buf_a, r_last)
    wait(r_last + 1, buf_b, sem_b)
    reduce_row(buf_b, r_last + 1)

    pltpu.sync_copy(
        out_v, out_hbm.at[pl.ds(w * (_ROWS_PER_W * _H), _ROWS_PER_W * _H)])

  return pool(idx_flat, g)


def _mlp_body(enc_ref, tl_ref, b1_ref, w2_ref, b2_ref, out_ref):
  h = enc_ref[...] * (1.0 / tl_ref[...]) + b1_ref[...]
  h = jnp.maximum(h, 0.0)
  out = lax.dot_general(w2_ref[...], h, (((1,), (1,)), ((), ())),
                        preferred_element_type=jnp.float32)
  out_ref[...] = out + b2_ref[...]


def _mlp(pooled, text_len, b1, w2, b2):
  bb = 512
  cc = w2.shape[0]
  return pl.pallas_call(
      _mlp_body,
      grid=(_B // bb,),
      in_specs=[
          pl.BlockSpec((bb, _H), lambda i: (i, 0)),
          pl.BlockSpec((bb, 1), lambda i: (i, 0)),
          pl.BlockSpec((1, _H), lambda i: (0, 0)),
          pl.BlockSpec((cc, _H), lambda i: (0, 0)),
          pl.BlockSpec((cc, 1), lambda i: (0, 0)),
      ],
      out_specs=pl.BlockSpec((cc, bb), lambda i: (0, i)),
      out_shape=jax.ShapeDtypeStruct((cc, _B), jnp.float32),
  )(pooled, text_len.reshape(_B, 1), b1.reshape(1, _H), w2,
    b2.reshape(cc, 1))


def kernel(input_text, text_len, table, W1, b1, W2, b2):
  g = _g_matmul(table.T, W1.T)
  idx_flat = input_text.reshape(_B * _L)
  pooled = _sc_pool(idx_flat, g).reshape(_B, _H)
  return _mlp(pooled, text_len, b1, W2, b2).T
